# baseline (speedup 1.0000x reference)
import jax
import jax.numpy as jnp
from jax.experimental import pallas as pl

_B, _N, _F = 16, 4096, 512
_E = 64
_ROWS = 64
_NBLK = _N // _ROWS


def _unpool_kernel(a_ref, lo_ref, hi_ref, o_ref):
    j = pl.program_id(1)
    body = a_ref[...]
    tail = 0.5 * (lo_ref[...] + hi_ref[...])
    o_ref[...] = jnp.where(j == _NBLK, tail, body)


def kernel(inputs):
    grid = (_B, _NBLK + 1)
    blk = (1, _ROWS, _F)
    out = pl.pallas_call(
        _unpool_kernel,
        grid=grid,
        in_specs=[
            pl.BlockSpec(blk, lambda b, j: (b, jnp.minimum(j, _NBLK - 1), 0)),
            pl.BlockSpec(blk, lambda b, j: (b, 0, 0)),
            pl.BlockSpec(blk, lambda b, j: (b, 2048 // _ROWS, 0)),
        ],
        out_specs=pl.BlockSpec(blk, lambda b, j: (b, j, 0)),
        out_shape=jax.ShapeDtypeStruct((_B, _N + _E, _F), inputs.dtype),
    )(inputs, inputs, inputs)
    return out
